# baseline (speedup 1.0000x reference)
import functools

import jax
import jax.numpy as jnp
from jax import lax
from jax.experimental import pallas as pl
from jax.experimental.pallas import tpu as pltpu
from jax.experimental.pallas import tpu_sc as plsc

BATCH = 16384
D = 64
LANES = 16
NC = 2
NS = 16
NW = NC * NS
B_PER_W = BATCH // NW
IDX_CHUNK = 128
N_CHUNKS = B_PER_W // IDX_CHUNK


def _gmf_body(ut_hbm, it_hbm, uidx_hbm, iidx_hbm, wb_hbm, out_hbm,
              uidx_v, iidx_v, urows_v, irows_v, wb_v, out_v, scan_v, sem):
    wid = lax.axis_index("s") * NC + lax.axis_index("c")
    base = wid * B_PER_W

    for j in range(N_CHUNKS):
        pltpu.sync_copy(uidx_hbm.at[pl.ds(base + j * IDX_CHUNK, IDX_CHUNK)],
                        uidx_v.at[j])
        pltpu.sync_copy(iidx_hbm.at[pl.ds(base + j * IDX_CHUNK, IDX_CHUNK)],
                        iidx_v.at[j])
    pltpu.sync_copy(wb_hbm, wb_v)

    copies = []
    for j in range(N_CHUNKS):
        copies.append(pltpu.async_copy(
            ut_hbm.at[uidx_v.at[j]],
            urows_v.at[pl.ds(j * IDX_CHUNK, IDX_CHUNK)], sem))
        copies.append(pltpu.async_copy(
            it_hbm.at[iidx_v.at[j]],
            irows_v.at[pl.ds(j * IDX_CHUNK, IDX_CHUNK)], sem))
    for c in copies:
        c.wait()

    w0 = wb_v[pl.ds(0, LANES)]
    w1 = wb_v[pl.ds(16, LANES)]
    w2 = wb_v[pl.ds(32, LANES)]
    w3 = wb_v[pl.ds(48, LANES)]
    bvec = wb_v[pl.ds(D, LANES)]
    lane = lax.iota(jnp.int32, LANES)
    last = jnp.full((LANES,), LANES - 1, jnp.int32)

    def group_body(g, carry):
        gbase = g * LANES
        for r in range(LANES):
            u = urows_v.at[gbase + r]
            it = irows_v.at[gbase + r]
            p = (u[pl.ds(0, LANES)] * it[pl.ds(0, LANES)] * w0
                 + u[pl.ds(16, LANES)] * it[pl.ds(16, LANES)] * w1
                 + u[pl.ds(32, LANES)] * it[pl.ds(32, LANES)] * w2
                 + u[pl.ds(48, LANES)] * it[pl.ds(48, LANES)] * w3)
            scan_v[r] = plsc.cumsum(p)
        sums = plsc.load_gather(scan_v, [lane, last])
        x = sums + bvec
        out_v[pl.ds(gbase, LANES)] = 1.0 / (1.0 + jnp.exp(-x))
        return carry

    lax.fori_loop(0, B_PER_W // LANES, group_body, 0)

    pltpu.sync_copy(out_v, out_hbm.at[pl.ds(base, B_PER_W)])


@jax.jit
def _gmf(user_indices, item_indices, user_table, item_table, wb):
    mesh = plsc.VectorSubcoreMesh(core_axis_name="c", subcore_axis_name="s")
    run = pl.kernel(
        _gmf_body,
        out_type=jax.ShapeDtypeStruct((BATCH,), jnp.float32),
        mesh=mesh,
        compiler_params=pltpu.CompilerParams(
            needs_layout_passes=False, use_tc_tiling_on_sc=False),
        scratch_types=[
            pltpu.VMEM((N_CHUNKS, IDX_CHUNK), jnp.int32),
            pltpu.VMEM((N_CHUNKS, IDX_CHUNK), jnp.int32),
            pltpu.VMEM((B_PER_W, D), jnp.float32),
            pltpu.VMEM((B_PER_W, D), jnp.float32),
            pltpu.VMEM((D + LANES,), jnp.float32),
            pltpu.VMEM((B_PER_W,), jnp.float32),
            pltpu.VMEM((LANES, LANES), jnp.float32),
            pltpu.SemaphoreType.DMA,
        ],
    )
    return run(user_table, item_table, user_indices, item_indices, wb)


def kernel(user_indices, item_indices, user_table, item_table, W, b):
    wb = jnp.concatenate(
        [W.reshape(-1), jnp.broadcast_to(b.reshape(-1), (LANES,))])
    return _gmf(user_indices.astype(jnp.int32), item_indices.astype(jnp.int32),
                user_table, item_table, wb)


# speedup vs baseline: 1.5278x; 1.5278x over previous
import jax
import jax.numpy as jnp
from jax import lax
from jax.experimental import pallas as pl
from jax.experimental.pallas import tpu as pltpu
from jax.experimental.pallas import tpu_sc as plsc

BATCH = 16384
D = 64
LANES = 16
NC = 2
NS = 16
NW = NC * NS
B_PER_W = BATCH // NW
CH = 256
K = 16


def _gmf_body(ut_hbm, it_hbm, uidx_hbm, iidx_hbm, wb_hbm, out_hbm,
              uidx_v, iidx_v, urows_v, irows_v, wb_v, out_v,
              scan_v, sem):
    wid = lax.axis_index("s") * NC + lax.axis_index("c")
    base = wid * B_PER_W

    pltpu.sync_copy(uidx_hbm.at[pl.ds(base, B_PER_W)], uidx_v)
    pltpu.sync_copy(iidx_hbm.at[pl.ds(base, B_PER_W)], iidx_v)
    pltpu.sync_copy(wb_hbm, wb_v)

    w0 = wb_v[pl.ds(0, LANES)]
    w1 = wb_v[pl.ds(16, LANES)]
    w2 = wb_v[pl.ds(32, LANES)]
    w3 = wb_v[pl.ds(48, LANES)]
    bvec = wb_v[pl.ds(D, LANES)]
    lane = lax.iota(jnp.int32, LANES)
    last = jnp.full((LANES,), LANES - 1, jnp.int32)

    for c in range(B_PER_W // CH):
        cbase = c * CH

        def fire_batch(j, carry):
            rb = cbase + j * K
            uvec = uidx_v[pl.ds(rb, K)]
            ivec = iidx_v[pl.ds(rb, K)]
            for r in range(K):
                pltpu.async_copy(ut_hbm.at[uvec[r]],
                                 urows_v.at[j * K + r], sem)
                pltpu.async_copy(it_hbm.at[ivec[r]],
                                 irows_v.at[j * K + r], sem)
            for r in range(K):
                pltpu.make_async_copy(ut_hbm.at[uvec[r]],
                                      urows_v.at[j * K + r], sem).wait()
                pltpu.make_async_copy(it_hbm.at[ivec[r]],
                                      irows_v.at[j * K + r], sem).wait()
            return carry

        lax.fori_loop(0, CH // K, fire_batch, 0, unroll=False)

        def group_body(g, carry):
            gbase = g * LANES
            for r in range(LANES):
                u = urows_v.at[gbase + r]
                it = irows_v.at[gbase + r]
                p = (u[pl.ds(0, LANES)] * it[pl.ds(0, LANES)] * w0
                     + u[pl.ds(16, LANES)] * it[pl.ds(16, LANES)] * w1
                     + u[pl.ds(32, LANES)] * it[pl.ds(32, LANES)] * w2
                     + u[pl.ds(48, LANES)] * it[pl.ds(48, LANES)] * w3)
                scan_v[r] = plsc.cumsum(p)
            sums = plsc.load_gather(scan_v, [lane, last])
            x = sums + bvec
            out_v[pl.ds(cbase + gbase, LANES)] = 1.0 / (1.0 + jnp.exp(-x))
            return carry

        lax.fori_loop(0, CH // LANES, group_body, 0, unroll=False)

    pltpu.sync_copy(out_v, out_hbm.at[pl.ds(base, B_PER_W)])


@jax.jit
def _gmf(user_indices, item_indices, user_table, item_table, wb):
    mesh = plsc.VectorSubcoreMesh(core_axis_name="c", subcore_axis_name="s")
    run = pl.kernel(
        _gmf_body,
        out_type=jax.ShapeDtypeStruct((BATCH,), jnp.float32),
        mesh=mesh,
        compiler_params=pltpu.CompilerParams(
            needs_layout_passes=False, use_tc_tiling_on_sc=True),
        scratch_types=[
            pltpu.VMEM((B_PER_W,), jnp.int32),
            pltpu.VMEM((B_PER_W,), jnp.int32),
            pltpu.VMEM((CH, D), jnp.float32),
            pltpu.VMEM((CH, D), jnp.float32),
            pltpu.VMEM((D + LANES,), jnp.float32),
            pltpu.VMEM((B_PER_W,), jnp.float32),
            pltpu.VMEM((LANES, LANES), jnp.float32),
            pltpu.SemaphoreType.DMA,
        ],
    )
    return run(user_table, item_table, user_indices, item_indices, wb)


def kernel(user_indices, item_indices, user_table, item_table, W, b):
    wb = jnp.concatenate(
        [W.reshape(-1), jnp.broadcast_to(b.reshape(-1), (LANES,))])
    return _gmf(user_indices.astype(jnp.int32), item_indices.astype(jnp.int32),
                user_table, item_table, wb)
